# Initial kernel scaffold; baseline (speedup 1.0000x reference)
import jax
import jax.numpy as jnp
from jax.experimental import pallas as pl


def kernel(x, group, W, b):
    raise NotImplementedError("write your pallas kernel here")


# speedup vs baseline: 2.4860x; 2.4860x over previous
import jax
import jax.numpy as jnp
from jax.experimental import pallas as pl

N_TOKENS = 8192
D_MODEL = 4096
NUM_GROUPS = 16
BLOCK_N = 256


def _block_kernel(x_ref, g_ref, w_ref, b_ref, o_ref):
    xb = x_ref[...]
    w = w_ref[...]
    scores = jax.lax.dot_general(
        xb, w, (((1,), (1,)), ((), ())),
        preferred_element_type=jnp.float32)
    gid = g_ref[...]
    cols = jax.lax.broadcasted_iota(jnp.int32, (xb.shape[0], NUM_GROUPS), 1)
    onehot = (cols == gid).astype(jnp.float32)
    bias = b_ref[...]
    y = jnp.sum((scores + bias) * onehot, axis=1, keepdims=True)
    o_ref[...] = y


def kernel(x, group, W, b):
    g2 = group.astype(jnp.int32).reshape(N_TOKENS, 1)
    w2 = W.reshape(NUM_GROUPS, D_MODEL)
    b2 = b.reshape(1, NUM_GROUPS)
    grid = N_TOKENS // BLOCK_N
    out = pl.pallas_call(
        _block_kernel,
        grid=(grid,),
        in_specs=[
            pl.BlockSpec((BLOCK_N, D_MODEL), lambda i: (i, 0)),
            pl.BlockSpec((BLOCK_N, 1), lambda i: (i, 0)),
            pl.BlockSpec((NUM_GROUPS, D_MODEL), lambda i: (0, 0)),
            pl.BlockSpec((1, NUM_GROUPS), lambda i: (0, 0)),
        ],
        out_specs=pl.BlockSpec((BLOCK_N, 1), lambda i: (i, 0)),
        out_shape=jax.ShapeDtypeStruct((N_TOKENS, 1), jnp.float32),
    )(x, g2, w2, b2)
    return out
